# baseline (speedup 1.0000x reference)
import functools

import jax
import jax.numpy as jnp
from jax import lax
from jax.experimental import pallas as pl
from jax.experimental.pallas import tpu as pltpu
from jax.experimental.pallas import tpu_sc as plsc

N = 10000
N_PAD = 10240
E = 320000
C = 128
NC = 2
NS = 16
K = 80
EDGES_PER_CORE = E // NC
EDGES_PER_TILE = EDGES_PER_CORE // NS
CHUNKS = EDGES_PER_TILE // K
ROWS_PER_TILE = N_PAD // NS


NB = 3
NI = 2 * NB
FULL_GROUPS = (CHUNKS // NI) * NI


def _sc_body(xt_hbm, gidx_hbm, ridx_hbm, zrow_hbm, zcnt_hbm,
             sum_out, cnt_out, rows, gvs, rvs, cnt_local,
             acc_sh, gsems, isems, ssems):
    c = lax.axis_index("c")
    s = lax.axis_index("s")
    row0 = s * ROWS_PER_TILE
    ones16 = jnp.ones((16,), jnp.float32)
    tile_base = c * EDGES_PER_CORE + s * EDGES_PER_TILE

    pltpu.sync_copy(zcnt_hbm, cnt_local)
    pltpu.sync_copy(zrow_hbm, rows[0])
    for z in range(ROWS_PER_TILE // K):
        zoff = pl.multiple_of(row0 + z * K, 8)
        pltpu.sync_copy(rows[0], acc_sh.at[pl.ds(zoff, K)])

    plsc.subcore_barrier()

    def _start_idx(j, sl):
        base = pl.multiple_of(tile_base + j * K, 8)
        pltpu.async_copy(gidx_hbm.at[pl.ds(base, K)], gvs[sl], isems[sl])
        pltpu.async_copy(ridx_hbm.at[pl.ds(base, K)], rvs[sl], isems[sl])

    def _start_gather(j, sl, q):
        base = pl.multiple_of(tile_base + j * K, 8)
        pltpu.make_async_copy(gidx_hbm.at[pl.ds(base, K)],
                              gvs[sl], isems[sl]).wait()
        pltpu.make_async_copy(ridx_hbm.at[pl.ds(base, K)],
                              rvs[sl], isems[sl]).wait()
        pltpu.async_copy(xt_hbm.at[gvs[sl]], rows[q], gsems[q])

    def _wait_gather(sl, q):
        pltpu.make_async_copy(xt_hbm.at[gvs[sl]], rows[q], gsems[q]).wait()

    def _start_scatter(sl, q):
        pltpu.async_copy(rows[q], acc_sh.at[rvs[sl]], ssems[q], add=True)

    def _wait_scatter(sl, q):
        pltpu.make_async_copy(rows[q], acc_sh.at[rvs[sl]], ssems[q]).wait()

    def _counts(sl):
        for i in range(K // 16):
            plsc.addupdate_scatter(cnt_local, [rvs[sl][pl.ds(i * 16, 16)]],
                                   ones16)

    for j in range(NI):
        _start_idx(j, j)
    _start_gather(0, 0, 0)
    _start_gather(1, 1, 1)

    for j in range(NI):
        b = j
        _wait_gather(b % NI, b % NB)
        _start_scatter(b % NI, b % NB)
        _counts(b % NI)
        if j >= 1:
            _wait_scatter((b - 1) % NI, (b - 1) % NB)
            _start_idx(j + 5, (b + 5) % NI)
        _start_gather(j + 2, (b + 2) % NI, (b + 2) % NB)

    FULL_END = NI + ((CHUNKS - NI) // NI) * NI

    @pl.loop(NI, FULL_END, step=NI)
    def _grp(j0):
        for b in range(NI):
            j = j0 + b
            _wait_gather(b % NI, b % NB)
            _start_scatter(b % NI, b % NB)
            _counts(b % NI)
            _wait_scatter((b - 1) % NI, (b - 1) % NB)
            _start_idx(j + 5, (b + 5) % NI)

            @pl.when(j + 2 < CHUNKS)
            def _():
                _start_gather(j + 2, (b + 2) % NI, (b + 2) % NB)

    for j in range(FULL_END, CHUNKS):
        b = j % NI
        _wait_gather(b % NI, b % NB)
        _start_scatter(b % NI, b % NB)
        _counts(b % NI)
        _wait_scatter((b - 1) % NI, (b - 1) % NB)
        if j + 2 < CHUNKS:
            _start_gather(j + 2, (b + 2) % NI, (b + 2) % NB)
    _wait_scatter((CHUNKS - 1) % NI, (CHUNKS - 1) % NB)

    plsc.subcore_barrier()

    for z in range(ROWS_PER_TILE // K):
        zoff = pl.multiple_of(row0 + z * K, 8)
        pltpu.sync_copy(acc_sh.at[pl.ds(zoff, K)], rows[0])
        pltpu.sync_copy(rows[0], sum_out.at[c, pl.ds(zoff, K)])
    pltpu.sync_copy(cnt_local, cnt_out.at[c, s])


@functools.cache
def _sc_aggregate_fn():
    mesh = plsc.VectorSubcoreMesh(core_axis_name="c", subcore_axis_name="s",
                                  num_cores=NC, num_subcores=NS)
    return pl.kernel(
        _sc_body,
        out_type=(
            jax.ShapeDtypeStruct((NC, N_PAD, C), jnp.float32),
            jax.ShapeDtypeStruct((NC, NS, N), jnp.float32),
        ),
        mesh=mesh,
        compiler_params=pltpu.CompilerParams(needs_layout_passes=False),
        scratch_types=[
            [pltpu.VMEM((K, C), jnp.float32)] * NB,
            [pltpu.VMEM((K,), jnp.int32)] * NI,
            [pltpu.VMEM((K,), jnp.int32)] * NI,
            pltpu.VMEM((N,), jnp.float32),
            pltpu.VMEM_SHARED((N_PAD, C), jnp.float32),
            [pltpu.SemaphoreType.DMA] * NB,
            [pltpu.SemaphoreType.DMA] * NI,
            [pltpu.SemaphoreType.DMA] * NB,
        ],
    )


def _tc_body(x_ref, s_ref, cnt_ref, wv_ref, bv_ref, wn_ref, bn_ref,
             gm_ref, bt_ref, o_ref):
    x = x_ref[...]
    ssum = s_ref[0, :N, :] + s_ref[1, :N, :]
    cnt_row = jnp.sum(cnt_ref[...], axis=0, keepdims=True)
    denom = jnp.maximum(cnt_row, 1.0)

    aggsum = lax.dot_general(wn_ref[...], ssum, (((1,), (1,)), ((), ())),
                             preferred_element_type=jnp.float32)
    agg = aggsum / denom + jnp.where(cnt_row > 0.0, 1.0, 0.0) * bn_ref[...]

    fv = lax.dot_general(wv_ref[...], x, (((1,), (0,)), ((), ())),
                         preferred_element_type=jnp.float32)
    out = agg + fv + bv_ref[...]

    mu = jnp.mean(out, axis=1, keepdims=True)
    d = out - mu
    var = jnp.mean(d * d, axis=1, keepdims=True)
    out = d * lax.rsqrt(var + 1e-5)
    out = out * gm_ref[...] + bt_ref[...]
    o_ref[...] = jnp.where(out > 0.0, out, 0.3 * out)


_tc_fused = pl.pallas_call(
    _tc_body,
    out_shape=jax.ShapeDtypeStruct((C, N), jnp.float32),
)


def kernel(in_features, reduce_index, gather_index, Wv, bv, Wn, bn, gamma, beta):
    x = in_features[0]
    xt = jnp.transpose(x)
    zrow = jnp.zeros((K, C), jnp.float32)
    zcnt = jnp.zeros((N,), jnp.float32)
    ssum, cntp = _sc_aggregate_fn()(xt, gather_index, reduce_index,
                                    zrow, zcnt)
    out = _tc_fused(x, ssum, cntp.reshape(NC * NS, N), Wv,
                    bv.reshape(C, 1), Wn, bn.reshape(C, 1),
                    gamma.reshape(C, 1), beta.reshape(C, 1))
    return out[None]


# speedup vs baseline: 1.0322x; 1.0322x over previous
import functools

import jax
import jax.numpy as jnp
from jax import lax
from jax.experimental import pallas as pl
from jax.experimental.pallas import tpu as pltpu
from jax.experimental.pallas import tpu_sc as plsc

N = 10000
N_PAD = 10240
E = 320000
C = 128
NC = 2
NS = 16
K = 80
EDGES_PER_CORE = E // NC
EDGES_PER_TILE = EDGES_PER_CORE // NS
CHUNKS = EDGES_PER_TILE // K
ROWS_PER_TILE = N_PAD // NS


NB = 3
NI = 2 * NB


def _sc_body(xt_hbm, gidx_hbm, ridx_hbm, zrow_hbm, zcnt_hbm,
             sum_out, cnt_out, rows, gvs, rvs, cnt_local,
             acc_sh, gsems, isems):
    c = lax.axis_index("c")
    s = lax.axis_index("s")
    row0 = s * ROWS_PER_TILE
    ones16 = jnp.ones((16,), jnp.float32)
    tile_base = c * EDGES_PER_CORE + s * EDGES_PER_TILE

    pltpu.sync_copy(zcnt_hbm, cnt_local)
    pltpu.sync_copy(zrow_hbm, rows[0])
    for z in range(ROWS_PER_TILE // K):
        zoff = pl.multiple_of(row0 + z * K, 8)
        pltpu.sync_copy(rows[0], acc_sh.at[pl.ds(zoff, K)])

    plsc.subcore_barrier()

    def _start_idx(j, sl):
        base = pl.multiple_of(tile_base + j * K, 8)
        pltpu.async_copy(gidx_hbm.at[pl.ds(base, K)], gvs[sl], isems[sl])
        pltpu.async_copy(ridx_hbm.at[pl.ds(base, K)], rvs[sl], isems[sl])

    def _start_gather(j, sl, q):
        base = pl.multiple_of(tile_base + j * K, 8)
        pltpu.make_async_copy(gidx_hbm.at[pl.ds(base, K)],
                              gvs[sl], isems[sl]).wait()
        pltpu.make_async_copy(ridx_hbm.at[pl.ds(base, K)],
                              rvs[sl], isems[sl]).wait()
        pltpu.async_copy(xt_hbm.at[gvs[sl]], rows[q], gsems[q])

    def _consume(sl, q):
        pltpu.make_async_copy(xt_hbm.at[gvs[sl]], rows[q], gsems[q]).wait()
        pltpu.sync_copy(rows[q], acc_sh.at[rvs[sl]], add=True)
        for i in range(K // 16):
            plsc.addupdate_scatter(cnt_local, [rvs[sl][pl.ds(i * 16, 16)]],
                                   ones16)

    for j in range(NI):
        _start_idx(j, j)
    for j in range(NB):
        _start_gather(j, j, j)

    FULL_GROUPS = (CHUNKS // NI) * NI

    @pl.loop(0, FULL_GROUPS, step=NI)
    def _grp(j0):
        for b in range(NI):
            j = j0 + b
            _consume(b % NI, b % NB)

            @pl.when(j + NI < CHUNKS)
            def _():
                _start_idx(j + NI, b % NI)
            _start_gather(j + NB, (b + NB) % NI, b % NB)

    for j in range(FULL_GROUPS, CHUNKS):
        _consume(j % NI, j % NB)
        if j + NB < CHUNKS:
            _start_gather(j + NB, (j + NB) % NI, (j + NB) % NB)

    plsc.subcore_barrier()

    for z in range(ROWS_PER_TILE // K):
        zoff = pl.multiple_of(row0 + z * K, 8)
        pltpu.sync_copy(acc_sh.at[pl.ds(zoff, K)], rows[0])
        pltpu.sync_copy(rows[0], sum_out.at[c, pl.ds(zoff, K)])
    pltpu.sync_copy(cnt_local, cnt_out.at[c, s])


@functools.cache
def _sc_aggregate_fn():
    mesh = plsc.VectorSubcoreMesh(core_axis_name="c", subcore_axis_name="s",
                                  num_cores=NC, num_subcores=NS)
    return pl.kernel(
        _sc_body,
        out_type=(
            jax.ShapeDtypeStruct((NC, N_PAD, C), jnp.float32),
            jax.ShapeDtypeStruct((NC, NS, N), jnp.float32),
        ),
        mesh=mesh,
        compiler_params=pltpu.CompilerParams(needs_layout_passes=False),
        scratch_types=[
            [pltpu.VMEM((K, C), jnp.float32)] * NB,
            [pltpu.VMEM((K,), jnp.int32)] * NI,
            [pltpu.VMEM((K,), jnp.int32)] * NI,
            pltpu.VMEM((N,), jnp.float32),
            pltpu.VMEM_SHARED((N_PAD, C), jnp.float32),
            [pltpu.SemaphoreType.DMA] * NB,
            [pltpu.SemaphoreType.DMA] * NI,
        ],
    )


def _tc_body(x_ref, s_ref, cnt_ref, wv_ref, bv_ref, wn_ref, bn_ref,
             gm_ref, bt_ref, o_ref):
    x = x_ref[...]
    ssum = s_ref[0, :N, :] + s_ref[1, :N, :]
    cnt_row = jnp.sum(cnt_ref[...], axis=0, keepdims=True)
    denom = jnp.maximum(cnt_row, 1.0)

    aggsum = lax.dot_general(wn_ref[...], ssum, (((1,), (1,)), ((), ())),
                             preferred_element_type=jnp.float32)
    agg = aggsum / denom + jnp.where(cnt_row > 0.0, 1.0, 0.0) * bn_ref[...]

    fv = lax.dot_general(wv_ref[...], x, (((1,), (0,)), ((), ())),
                         preferred_element_type=jnp.float32)
    out = agg + fv + bv_ref[...]

    mu = jnp.mean(out, axis=1, keepdims=True)
    d = out - mu
    var = jnp.mean(d * d, axis=1, keepdims=True)
    out = d * lax.rsqrt(var + 1e-5)
    out = out * gm_ref[...] + bt_ref[...]
    o_ref[...] = jnp.where(out > 0.0, out, 0.3 * out)


_tc_fused = pl.pallas_call(
    _tc_body,
    out_shape=jax.ShapeDtypeStruct((C, N), jnp.float32),
)


def kernel(in_features, reduce_index, gather_index, Wv, bv, Wn, bn, gamma, beta):
    x = in_features[0]
    xt = jnp.transpose(x)
    zrow = jnp.zeros((K, C), jnp.float32)
    zcnt = jnp.zeros((N,), jnp.float32)
    ssum, cntp = _sc_aggregate_fn()(xt, gather_index, reduce_index,
                                    zrow, zcnt)
    out = _tc_fused(x, ssum, cntp.reshape(NC * NS, N), Wv,
                    bv.reshape(C, 1), Wn, bn.reshape(C, 1),
                    gamma.reshape(C, 1), beta.reshape(C, 1))
    return out[None]
